# Initial kernel scaffold; baseline (speedup 1.0000x reference)
import jax
import jax.numpy as jnp
from jax.experimental import pallas as pl


def kernel(x, token_table, pos_table):
    raise NotImplementedError("write your pallas kernel here")


# speedup vs baseline: 1.3197x; 1.3197x over previous
import functools

import jax
import jax.numpy as jnp
from jax import lax
from jax.experimental import pallas as pl
from jax.experimental.pallas import tpu as pltpu
from jax.experimental.pallas import tpu_sc as plsc

MAXLEN = 200
EMBED_DIM = 32
NUM_WORKERS = 32
SEQ_PER_CHUNK = 8
CHUNK = SEQ_PER_CHUNK * MAXLEN


def _gather_slices(chunk):
    slices = []
    off = 0
    while off < chunk:
        size = min(128, chunk - off)
        slices.append((off, size))
        off += size
    return slices


@functools.partial(jax.jit, static_argnames=("n_rows",))
def _embed(x_flat, token_table, pos_table, n_rows):
    per_w = n_rows // NUM_WORKERS
    n_chunks = per_w // CHUNK
    slices = _gather_slices(CHUNK)
    mesh = plsc.VectorSubcoreMesh(core_axis_name="c", subcore_axis_name="s")

    @functools.partial(
        pl.kernel,
        mesh=mesh,
        out_type=jax.ShapeDtypeStruct((n_rows, EMBED_DIM), jnp.float32),
        scratch_types=[
            pltpu.VMEM((CHUNK,), jnp.int32),
            pltpu.VMEM((CHUNK, EMBED_DIM), jnp.float32),
            pltpu.VMEM((MAXLEN, EMBED_DIM), jnp.float32),
            pltpu.SemaphoreType.DMA,
        ],
        compiler_params=pltpu.CompilerParams(use_tc_tiling_on_sc=False),
    )
    def k(x_hbm, tok_hbm, pos_hbm, out_hbm, idx_v, rows_v, pos_v, sem):
        wid = lax.axis_index("s") * 2 + lax.axis_index("c")
        base = wid * per_w
        pltpu.sync_copy(pos_hbm, pos_v)

        def chunk_body(c, carry):
            off = base + c * CHUNK
            pltpu.sync_copy(x_hbm.at[pl.ds(off, CHUNK)], idx_v)
            for s_off, s_size in slices:
                pltpu.async_copy(
                    tok_hbm.at[idx_v.at[pl.ds(s_off, s_size)]],
                    rows_v.at[pl.ds(s_off, s_size)],
                    sem,
                )
            for s_off, s_size in slices:
                pltpu.make_async_copy(
                    tok_hbm.at[idx_v.at[pl.ds(s_off, s_size)]],
                    rows_v.at[pl.ds(s_off, s_size)],
                    sem,
                ).wait()

            def seq_body(s, carry2):
                def row_body(p, carry3):
                    r = s * MAXLEN + p
                    rows_v[r, pl.ds(0, 16)] = (
                        rows_v[r, pl.ds(0, 16)] + pos_v[p, pl.ds(0, 16)]
                    )
                    rows_v[r, pl.ds(16, 16)] = (
                        rows_v[r, pl.ds(16, 16)] + pos_v[p, pl.ds(16, 16)]
                    )
                    return carry3

                return lax.fori_loop(0, MAXLEN, row_body, carry2)

            lax.fori_loop(0, SEQ_PER_CHUNK, seq_body, 0)
            pltpu.sync_copy(rows_v, out_hbm.at[pl.ds(off, CHUNK)])
            return carry

        lax.fori_loop(0, n_chunks, chunk_body, 0)

    return k(x_flat, token_table, pos_table)


def kernel(x, token_table, pos_table):
    batch, maxlen = x.shape
    n_rows = batch * maxlen
    x_flat = x.reshape(n_rows).astype(jnp.int32)
    out = _embed(x_flat, token_table, pos_table, n_rows)
    return out.reshape(batch, maxlen, EMBED_DIM)
